# baseline (speedup 1.0000x reference)
import functools

import jax
import jax.numpy as jnp
from jax import lax
from jax.experimental import pallas as pl
from jax.experimental.pallas import tpu as pltpu
from jax.experimental.pallas import tpu_sc as plsc

_L = 16


def _make_sc_combine(N, K, D, rows_per_step, n_workers):
    rpw = N // n_workers
    R = rows_per_step
    nsteps = rpw // R
    ngroups = nsteps // 2
    nchunks = D // _L

    mesh = plsc.VectorSubcoreMesh(core_axis_name="c", subcore_axis_name="s")
    info = plsc.get_sparse_core_info()
    nc = info.num_cores

    @functools.partial(
        pl.kernel,
        mesh=mesh,
        out_type=jax.ShapeDtypeStruct((N, D), jnp.float32),
        scratch_types=[
            pltpu.VMEM((rpw * K * _L,), jnp.float32),
            pltpu.VMEM((R, K, D), jnp.float32),
            pltpu.VMEM((R, K, D), jnp.float32),
            pltpu.VMEM((R, D), jnp.float32),
            pltpu.VMEM((R, D), jnp.float32),
            pltpu.SemaphoreType.DMA,
            pltpu.SemaphoreType.DMA,
            pltpu.SemaphoreType.DMA,
            pltpu.SemaphoreType.DMA,
        ],
    )
    def sc_combine(x_hbm, w_hbm, out_hbm, w_v, in0, in1, o0, o1,
                   si0, si1, so0, so1):
        wid = lax.axis_index("s") * nc + lax.axis_index("c")
        base = wid * rpw

        pltpu.sync_copy(w_hbm.at[pl.ds(base * K * _L, rpw * K * _L)], w_v)

        def start_in(step, buf, sem):
            pltpu.async_copy(x_hbm.at[pl.ds(base + step * R, R)], buf, sem)

        def start_out(step, buf, sem):
            pltpu.async_copy(buf, out_hbm.at[pl.ds(base + step * R, R)], sem)

        start_in(0, in0, si0)
        start_in(1, in1, si1)

        bufs = ((in0, o0, si0, so0), (in1, o1, si1, so1))

        def group(g, carry):
            for b in range(2):
                ib, ob, isem, osem = bufs[b]
                step = g * 2 + b
                pltpu.make_async_copy(
                    x_hbm.at[pl.ds(0, R)], ib, isem).wait()

                @pl.when(g > 0)
                def _drain_out():
                    pltpu.make_async_copy(
                        ob, out_hbm.at[pl.ds(0, R)], osem).wait()

                w_row = step * R
                ws = [(w_v[pl.ds((w_row + r) * K * _L, _L)],
                       w_v[pl.ds((w_row + r) * K * _L + _L, _L)])
                      for r in range(R)]

                def chunk(j, ccarry):
                    off = pl.multiple_of(j * _L, _L)
                    for r in range(R):
                        x0 = ib[r, 0, pl.ds(off, _L)]
                        x1 = ib[r, 1, pl.ds(off, _L)]
                        ob[r, pl.ds(off, _L)] = x0 * ws[r][0] + x1 * ws[r][1]
                    return ccarry

                lax.fori_loop(0, nchunks, chunk, None, unroll=False)

                start_out(step, ob, osem)

                @pl.when(g < ngroups - 1)
                def _prefetch_in():
                    start_in(step + 2, ib, isem)

            return carry

        lax.fori_loop(0, ngroups, group, None, unroll=False)

        pltpu.make_async_copy(o0, out_hbm.at[pl.ds(0, R)], so0).wait()
        pltpu.make_async_copy(o1, out_hbm.at[pl.ds(0, R)], so1).wait()

    return sc_combine


def kernel(combined_output, weights):
    B, T, K, D = combined_output.shape
    N = B * T
    x = combined_output.reshape(N, K, D)
    w = jnp.repeat(weights.reshape(N * K), _L)
    sc = _make_sc_combine(N, K, D, rows_per_step=8, n_workers=32)
    out = sc(x, w)
    return out.reshape(B, T, D)


# speedup vs baseline: 2.2654x; 2.2654x over previous
import functools

import jax
import jax.numpy as jnp
from jax import lax
from jax.experimental import pallas as pl
from jax.experimental.pallas import tpu as pltpu
from jax.experimental.pallas import tpu_sc as plsc

_L = 16


def _make_sc_combine(N, K, D, rows_per_step, n_workers):
    rpw = N // n_workers
    R = rows_per_step
    nsteps = rpw // R
    ngroups = nsteps // 2
    nchunks = D // _L

    mesh = plsc.VectorSubcoreMesh(core_axis_name="c", subcore_axis_name="s")
    info = plsc.get_sparse_core_info()
    nc = info.num_cores

    @functools.partial(
        pl.kernel,
        mesh=mesh,
        out_type=jax.ShapeDtypeStruct((N, D), jnp.float32),
        scratch_types=[
            pltpu.VMEM((rpw * K * _L,), jnp.float32),
            pltpu.VMEM((R, K, D), jnp.float32),
            pltpu.VMEM((R, K, D), jnp.float32),
            pltpu.VMEM((R, D), jnp.float32),
            pltpu.VMEM((R, D), jnp.float32),
            pltpu.SemaphoreType.DMA,
            pltpu.SemaphoreType.DMA,
            pltpu.SemaphoreType.DMA,
            pltpu.SemaphoreType.DMA,
        ],
    )
    def sc_combine(x_hbm, w_hbm, out_hbm, w_v, in0, in1, o0, o1,
                   si0, si1, so0, so1):
        wid = lax.axis_index("s") * nc + lax.axis_index("c")
        base = wid * rpw

        pltpu.sync_copy(w_hbm.at[pl.ds(base * K * _L, rpw * K * _L)], w_v)

        def start_in(step, buf, sem):
            pltpu.async_copy(x_hbm.at[pl.ds(base + step * R, R)], buf, sem)

        def start_out(step, buf, sem):
            pltpu.async_copy(buf, out_hbm.at[pl.ds(base + step * R, R)], sem)

        start_in(0, in0, si0)
        start_in(1, in1, si1)

        bufs = ((in0, o0, si0, so0), (in1, o1, si1, so1))

        def group(g, carry):
            for b in range(2):
                ib, ob, isem, osem = bufs[b]
                step = g * 2 + b
                pltpu.make_async_copy(
                    x_hbm.at[pl.ds(0, R)], ib, isem).wait()

                @pl.when(g > 0)
                def _drain_out():
                    pltpu.make_async_copy(
                        ob, out_hbm.at[pl.ds(0, R)], osem).wait()

                w_row = step * R
                ws = [(w_v[pl.ds((w_row + r) * K * _L, _L)],
                       w_v[pl.ds((w_row + r) * K * _L + _L, _L)])
                      for r in range(R)]

                @plsc.parallel_loop(0, D, step=_L, unroll=2)
                def _chunk(off):
                    for r in range(R):
                        x0 = ib[r, 0, pl.ds(off, _L)]
                        x1 = ib[r, 1, pl.ds(off, _L)]
                        ob[r, pl.ds(off, _L)] = x0 * ws[r][0] + x1 * ws[r][1]

                start_out(step, ob, osem)

                @pl.when(g < ngroups - 1)
                def _prefetch_in():
                    start_in(step + 2, ib, isem)

            return carry

        lax.fori_loop(0, ngroups, group, None, unroll=False)

        pltpu.make_async_copy(o0, out_hbm.at[pl.ds(0, R)], so0).wait()
        pltpu.make_async_copy(o1, out_hbm.at[pl.ds(0, R)], so1).wait()

    return sc_combine


def kernel(combined_output, weights):
    B, T, K, D = combined_output.shape
    N = B * T
    x = combined_output.reshape(N, K, D)
    w = jnp.repeat(weights.reshape(N * K), _L)
    sc = _make_sc_combine(N, K, D, rows_per_step=8, n_workers=32)
    out = sc(x, w)
    return out.reshape(B, T, D)


# speedup vs baseline: 2.5149x; 1.1101x over previous
import jax
import jax.numpy as jnp
from jax.experimental import pallas as pl
from jax.experimental.pallas import tpu as pltpu

_ROWS_PER_BLOCK = 512


def _combine_body(x_ref, w_ref, o_ref):
    x = x_ref[...]
    w = w_ref[...]
    o_ref[...] = x[:, 0, :] * w[:, 0:1] + x[:, 1, :] * w[:, 1:2]


def kernel(combined_output, weights):
    B, T, K, D = combined_output.shape
    N = B * T
    x = combined_output.reshape(N, K, D)
    w = weights.reshape(N, K)
    R = _ROWS_PER_BLOCK
    grid = (N // R,)
    out = pl.pallas_call(
        _combine_body,
        grid=grid,
        in_specs=[
            pl.BlockSpec((R, K, D), lambda i: (i, 0, 0)),
            pl.BlockSpec((R, K), lambda i: (i, 0)),
        ],
        out_specs=pl.BlockSpec((R, D), lambda i: (i, 0)),
        out_shape=jax.ShapeDtypeStruct((N, D), combined_output.dtype),
    )(x, w)
    return out.reshape(B, T, D)
